# Initial kernel scaffold; baseline (speedup 1.0000x reference)
import jax
import jax.numpy as jnp
from jax.experimental import pallas as pl


def kernel(state, quantization_keys, in_proj_weight, in_proj_bias, embed0, embed1, embed2):
    raise NotImplementedError("write your pallas kernel here")


# speedup vs baseline: 1.6218x; 1.6218x over previous
import functools

import jax
import jax.numpy as jnp
from jax import lax
from jax.experimental import pallas as pl
from jax.experimental.pallas import tpu as pltpu
from jax.experimental.pallas import tpu_sc as plsc

_N_FACTORS = (1, 2, 4)
_D = 256
_NE = 2730
_NEP = 2816
_H = 4
_DH = 64
_R = 256
_ALPHA = 0.01
_NLV = 3
_BSZ = 4096
_BIG = 1e30


def _tc_body(x_ref, keysp_ref, wqt_ref, wkt_ref, bq_ref, bk_ref, gn_ref,
             e0t_ref, e1t_ref, e2t_ref, att_ref, gidx_ref, extra_ref):
    i = pl.program_id(0)
    X = x_ref[...]
    lane8 = lax.broadcasted_iota(jnp.int32, (_R, 8), 1)

    q = jnp.dot(X, wqt_ref[...], preferred_element_type=jnp.float32) + bq_ref[...]
    k = jnp.dot(keysp_ref[...], wkt_ref[...], preferred_element_type=jnp.float32) + bk_ref[...]
    scale = jnp.float32(_DH ** -0.5)
    att_soft = jnp.zeros((_R, 8), jnp.float32)
    for h in range(_H):
        qh = q[:, h * _DH:(h + 1) * _DH] * scale
        kh = k[:, h * _DH:(h + 1) * _DH]
        lg = lax.dot_general(qh, kh, (((1,), (1,)), ((), ())),
                             preferred_element_type=jnp.float32)
        lg = jnp.where(lane8 < _NLV, lg, -jnp.inf)
        mx = jnp.max(lg, axis=1, keepdims=True)
        ex = jnp.exp(lg - mx)
        att_soft = att_soft + ex / jnp.sum(ex, axis=1, keepdims=True)
    att_soft = att_soft * jnp.float32(1.0 / _H)

    yl = att_soft + gn_ref[...]
    mx = jnp.max(yl, axis=1, keepdims=True)
    ex = jnp.exp(yl - mx)
    y = ex / jnp.sum(ex, axis=1, keepdims=True)
    ymax = jnp.max(y, axis=1, keepdims=True)
    sel = jnp.min(jnp.where(y == ymax, lane8, 1 << 30), axis=1, keepdims=True)
    y_hard = jnp.where(lane8 == sel, jnp.float32(1.0), jnp.float32(0.0))
    att_out = (y_hard - y) + y
    att_ref[...] = att_out

    lane_c = lax.broadcasted_iota(jnp.int32, (1, _NEP), 1)
    pad_pen = jnp.where(lane_c < _NE, jnp.float32(0.0), jnp.float32(_BIG))

    def level(Xs, ET):
        x2 = jnp.sum(Xs * Xs, axis=1, keepdims=True)
        e2 = jnp.sum(ET * ET, axis=0, keepdims=True) + pad_pen
        m = jnp.dot(Xs, ET, preferred_element_type=jnp.float32)
        dist = (x2 - 2.0 * m) + e2
        minv = jnp.min(dist, axis=1, keepdims=True)
        lanes = lax.broadcasted_iota(jnp.int32, dist.shape, 1)
        ind = jnp.min(jnp.where(dist == minv, lanes, 1 << 30), axis=1,
                      keepdims=True)
        return ind, jnp.sum(minv)

    ind0, s0 = level(X, e0t_ref[...])
    E1 = e1t_ref[...]
    ind1 = []
    s1 = jnp.float32(0.0)
    for sb in range(2):
        ind, s = level(X[:, sb * 128:(sb + 1) * 128], E1)
        ind1.append(ind)
        s1 = s1 + s
    E2 = e2t_ref[...]
    ind2 = []
    s2 = jnp.float32(0.0)
    for sb in range(4):
        ind, s = level(X[:, sb * 64:(sb + 1) * 64], E2)
        ind2.append(ind)
        s2 = s2 + s

    g0 = ind0 * 4 + lane8
    g1 = (_NE * 4) + jnp.where(lane8 < 2, ind1[0], ind1[1]) * 2 + (lane8 & 1)
    c01 = jnp.where(lane8 == 0, ind2[0], ind2[1])
    c23 = jnp.where(lane8 == 2, ind2[2], ind2[3])
    g2 = (_NE * 6) + jnp.where(lane8 < 2, c01, c23)
    g = jnp.where(sel == 0, g0, jnp.where(sel == 1, g1, g2))
    gidx_ref[...] = jnp.where(lane8 < 4, g, 0)

    nvec = jnp.where(lane8 == 0, jnp.float32(1.0),
                     jnp.where(lane8 == 1, jnp.float32(2.0),
                               jnp.where(lane8 == 2, jnp.float32(4.0),
                                         jnp.float32(0.0))))
    pen_tile = jnp.sum(nvec * att_out)
    contrib = ((s0 + s1 + s2) * jnp.float32(1.0 / (_NLV * _BSZ * _D))
               + jnp.float32(_ALPHA / _BSZ) * pen_tile)

    @pl.when(i == 0)
    def _():
        extra_ref[...] = jnp.zeros((1, 1), jnp.float32)

    extra_ref[...] = extra_ref[...] + jnp.reshape(contrib, (1, 1))


def _tc_call(X, keysp, wqt, wkt, bq, bk, gn, e0t, e1t, e2t):
    n_tiles = _BSZ // _R
    full = lambda shape: pl.BlockSpec(shape, lambda i: (0, 0))
    return pl.pallas_call(
        _tc_body,
        grid=(n_tiles,),
        in_specs=[
            pl.BlockSpec((_R, _D), lambda i: (i, 0)),
            full((8, _D)),
            full((_D, _D)),
            full((_D, _D)),
            full((1, _D)),
            full((1, _D)),
            pl.BlockSpec((_R, 8), lambda i: (i, 0)),
            full((_D, _NEP)),
            full((128, _NEP)),
            full((64, _NEP)),
        ],
        out_specs=[
            pl.BlockSpec((_R, 8), lambda i: (i, 0)),
            pl.BlockSpec((_R, 8), lambda i: (i, 0)),
            pl.BlockSpec((1, 1), lambda i: (0, 0)),
        ],
        out_shape=[
            jax.ShapeDtypeStruct((_BSZ, 8), jnp.float32),
            jax.ShapeDtypeStruct((_BSZ, 8), jnp.int32),
            jax.ShapeDtypeStruct((1, 1), jnp.float32),
        ],
    )(X, keysp, wqt, wkt, bq, bk, gn, e0t, e1t, e2t)


def _sc_gather(table, gidx3):
    info = plsc.get_sparse_core_info()
    nc, ns = info.num_cores, info.num_subcores
    nw = nc * ns
    rows_per_w = 16384 // nw
    mesh = plsc.VectorSubcoreMesh(core_axis_name="c", subcore_axis_name="s")

    @functools.partial(
        pl.kernel,
        mesh=mesh,
        compiler_params=pltpu.CompilerParams(use_tc_tiling_on_sc=False),
        out_type=jax.ShapeDtypeStruct((16384, 64), jnp.float32),
        scratch_types=[
            pltpu.VMEM((4, 128), jnp.int32),
            pltpu.VMEM((rows_per_w, 64), jnp.float32),
            pltpu.SemaphoreType.DMA,
        ],
    )
    def k(table_hbm, idx_hbm, out_hbm, idx_v, rows_v, sem):
        wid = lax.axis_index("s") * nc + lax.axis_index("c")
        pltpu.sync_copy(idx_hbm.at[wid], idx_v)
        cps = [
            pltpu.async_copy(table_hbm.at[idx_v.at[j]],
                             rows_v.at[pl.ds(j * 128, 128)], sem)
            for j in range(4)
        ]
        for cp in cps:
            cp.wait()
        pltpu.sync_copy(rows_v, out_hbm.at[pl.ds(wid * rows_per_w, rows_per_w)])

    return k(table, gidx3)


def kernel(state, quantization_keys, in_proj_weight, in_proj_bias,
           embed0, embed1, embed2):
    bsz, T, Hsz = state.shape
    X = state.reshape(bsz, Hsz)
    wqt = in_proj_weight[:Hsz].T
    wkt = in_proj_weight[Hsz:2 * Hsz].T
    bq = in_proj_bias[:Hsz].reshape(1, Hsz)
    bk = in_proj_bias[Hsz:2 * Hsz].reshape(1, Hsz)
    keysp = jnp.zeros((8, Hsz), jnp.float32).at[:_NLV].set(
        quantization_keys.reshape(_NLV, Hsz))

    u = jax.random.uniform(jax.random.key(42), (1, bsz, _NLV),
                           minval=1e-10, maxval=1.0)
    g = -jnp.log(-jnp.log(u))[0]
    gn = jnp.concatenate(
        [g, jnp.full((bsz, 8 - _NLV), -jnp.inf, jnp.float32)], axis=1)

    pad = lambda e: jnp.pad(e, ((0, _NEP - _NE), (0, 0)))
    e0t = pad(embed0).T
    e1t = pad(embed1).T
    e2t = pad(embed2).T

    att8, gidx8, extra = _tc_call(X, keysp, wqt, wkt, bq, bk, gn,
                                  e0t, e1t, e2t)

    att = att8[:, :_NLV].reshape(1, bsz, _NLV)
    table = jnp.concatenate(
        [embed0.reshape(-1, 64), embed1.reshape(-1, 64),
         embed2.reshape(-1, 64)], axis=0)
    gidx3 = gidx8[:, :4].reshape(32, 4, 128)
    out_rows = _sc_gather(table, gidx3)
    out = out_rows.reshape(bsz, T, Hsz)
    return out, extra[0, 0], att


# speedup vs baseline: 2.5028x; 1.5433x over previous
import functools

import jax
import jax.numpy as jnp
from jax import lax
from jax.experimental import pallas as pl
from jax.experimental.pallas import tpu as pltpu
from jax.experimental.pallas import tpu_sc as plsc

_N_FACTORS = (1, 2, 4)
_D = 256
_NE = 2730
_NEP = 2816
_H = 4
_DH = 64
_R = 256
_ALPHA = 0.01
_NLV = 3
_BSZ = 4096
_BIG = 1e30


def _tc_body(x_ref, keysp_ref, wqt_ref, wkt_ref, bq_ref, bk_ref, gn_ref,
             e0t_ref, e1t_ref, e2t_ref, att_ref, gidx_ref, extra_ref,
             he2_ref):
    i = pl.program_id(0)
    X = x_ref[...]
    lane8 = lax.broadcasted_iota(jnp.int32, (_R, 8), 1)

    lane_c = lax.broadcasted_iota(jnp.int32, (1, _NEP), 1)

    @pl.when(i == 0)
    def _():
        pad_pen = jnp.where(lane_c < _NE, jnp.float32(0.0), jnp.float32(_BIG))
        for r, et_ref in ((0, e0t_ref), (1, e1t_ref), (2, e2t_ref)):
            ET = et_ref[...]
            he2_ref[r:r + 1, :] = (
                jnp.float32(0.5) * jnp.sum(ET * ET, axis=0, keepdims=True)
                + pad_pen)

    q = jnp.dot(X, wqt_ref[...], preferred_element_type=jnp.float32) + bq_ref[...]
    k = jnp.dot(keysp_ref[...], wkt_ref[...], preferred_element_type=jnp.float32) + bk_ref[...]
    scale = jnp.float32(_DH ** -0.5)
    att_soft = jnp.zeros((_R, 8), jnp.float32)
    for h in range(_H):
        qh = q[:, h * _DH:(h + 1) * _DH] * scale
        kh = k[:, h * _DH:(h + 1) * _DH]
        lg = lax.dot_general(qh, kh, (((1,), (1,)), ((), ())),
                             preferred_element_type=jnp.float32)
        lg = jnp.where(lane8 < _NLV, lg, -jnp.inf)
        mx = jnp.max(lg, axis=1, keepdims=True)
        ex = jnp.exp(lg - mx)
        att_soft = att_soft + ex / jnp.sum(ex, axis=1, keepdims=True)
    att_soft = att_soft * jnp.float32(1.0 / _H)

    yl = att_soft + gn_ref[...]
    mx = jnp.max(yl, axis=1, keepdims=True)
    ex = jnp.exp(yl - mx)
    y = ex / jnp.sum(ex, axis=1, keepdims=True)
    ymax = jnp.max(y, axis=1, keepdims=True)
    sel = jnp.min(jnp.where(y == ymax, lane8, 1 << 30), axis=1, keepdims=True)
    y_hard = jnp.where(lane8 == sel, jnp.float32(1.0), jnp.float32(0.0))
    att_out = (y_hard - y) + y
    att_ref[...] = att_out

    _W = 128
    _NCH = _NEP // _W
    lane_w = lax.broadcasted_iota(jnp.int32, (_R, _W), 1)

    _RH = 64
    lane_f = lax.broadcasted_iota(jnp.int32, (_RH, _W), 1).astype(jnp.float32)

    def level(Xs, ET, lv):
        x2 = jnp.sum(Xs * Xs, axis=1, keepdims=True)
        he2 = he2_ref[lv:lv + 1, :]
        m = jnp.dot(Xs, ET, preferred_element_type=jnp.float32)
        inds, smaxs = [], []
        for rb in range(_R // _RH):
            ms = m[rb * _RH:(rb + 1) * _RH]
            sacc = ms[:, :_W] - he2[:, :_W]
            iacc = lane_f
            for c in range(1, _NCH):
                s = (ms[:, c * _W:(c + 1) * _W]
                     - he2[:, c * _W:(c + 1) * _W])
                iacc = jnp.where(s > sacc, lane_f + jnp.float32(c * _W), iacc)
                sacc = jnp.maximum(sacc, s)
            mrow = jnp.max(sacc, axis=1, keepdims=True)
            indf = jnp.min(jnp.where(sacc == mrow, iacc, jnp.float32(1e9)),
                           axis=1, keepdims=True)
            inds.append(indf.astype(jnp.int32))
            smaxs.append(mrow)
        ind = jnp.concatenate(inds, axis=0)
        smax = jnp.concatenate(smaxs, axis=0)
        return ind, jnp.sum(x2 - 2.0 * smax)

    ind0, s0 = level(X, e0t_ref[...], 0)
    E1 = e1t_ref[...]
    ind1 = []
    s1 = jnp.float32(0.0)
    for sb in range(2):
        ind, s = level(X[:, sb * 128:(sb + 1) * 128], E1, 1)
        ind1.append(ind)
        s1 = s1 + s
    E2 = e2t_ref[...]
    ind2 = []
    s2 = jnp.float32(0.0)
    for sb in range(4):
        ind, s = level(X[:, sb * 64:(sb + 1) * 64], E2, 2)
        ind2.append(ind)
        s2 = s2 + s

    g0 = ind0 * 4 + lane8
    g1 = (_NE * 4) + jnp.where(lane8 < 2, ind1[0], ind1[1]) * 2 + (lane8 & 1)
    c01 = jnp.where(lane8 == 0, ind2[0], ind2[1])
    c23 = jnp.where(lane8 == 2, ind2[2], ind2[3])
    g2 = (_NE * 6) + jnp.where(lane8 < 2, c01, c23)
    g = jnp.where(sel == 0, g0, jnp.where(sel == 1, g1, g2))
    gidx_ref[...] = jnp.where(lane8 < 4, g, 0)

    nvec = jnp.where(lane8 == 0, jnp.float32(1.0),
                     jnp.where(lane8 == 1, jnp.float32(2.0),
                               jnp.where(lane8 == 2, jnp.float32(4.0),
                                         jnp.float32(0.0))))
    pen_tile = jnp.sum(nvec * att_out)
    contrib = ((s0 + s1 + s2) * jnp.float32(1.0 / (_NLV * _BSZ * _D))
               + jnp.float32(_ALPHA / _BSZ) * pen_tile)

    @pl.when(i == 0)
    def _():
        extra_ref[...] = jnp.zeros((1, 1), jnp.float32)

    extra_ref[...] = extra_ref[...] + jnp.reshape(contrib, (1, 1))


def _tc_call(X, keysp, wqt, wkt, bq, bk, gn, e0t, e1t, e2t):
    n_tiles = _BSZ // _R
    full = lambda shape: pl.BlockSpec(shape, lambda i: (0, 0))
    return pl.pallas_call(
        _tc_body,
        grid=(n_tiles,),
        in_specs=[
            pl.BlockSpec((_R, _D), lambda i: (i, 0)),
            full((8, _D)),
            full((_D, _D)),
            full((_D, _D)),
            full((1, _D)),
            full((1, _D)),
            pl.BlockSpec((_R, 8), lambda i: (i, 0)),
            full((_D, _NEP)),
            full((128, _NEP)),
            full((64, _NEP)),
        ],
        out_specs=[
            pl.BlockSpec((_R, 8), lambda i: (i, 0)),
            pl.BlockSpec((_R, 8), lambda i: (i, 0)),
            pl.BlockSpec((1, 1), lambda i: (0, 0)),
        ],
        out_shape=[
            jax.ShapeDtypeStruct((_BSZ, 8), jnp.float32),
            jax.ShapeDtypeStruct((_BSZ, 8), jnp.int32),
            jax.ShapeDtypeStruct((1, 1), jnp.float32),
        ],
        scratch_shapes=[pltpu.VMEM((8, _NEP), jnp.float32)],
    )(X, keysp, wqt, wkt, bq, bk, gn, e0t, e1t, e2t)


def _sc_gather(table, gidx3):
    info = plsc.get_sparse_core_info()
    nc, ns = info.num_cores, info.num_subcores
    nw = nc * ns
    rows_per_w = 16384 // nw
    mesh = plsc.VectorSubcoreMesh(core_axis_name="c", subcore_axis_name="s")

    @functools.partial(
        pl.kernel,
        mesh=mesh,
        compiler_params=pltpu.CompilerParams(use_tc_tiling_on_sc=False),
        out_type=jax.ShapeDtypeStruct((16384, 64), jnp.float32),
        scratch_types=[
            pltpu.VMEM((4, 128), jnp.int32),
            pltpu.VMEM((rows_per_w, 64), jnp.float32),
            pltpu.SemaphoreType.DMA,
        ],
    )
    def k(table_hbm, idx_hbm, out_hbm, idx_v, rows_v, sem):
        wid = lax.axis_index("s") * nc + lax.axis_index("c")
        pltpu.sync_copy(idx_hbm.at[wid], idx_v)
        cps = [
            pltpu.async_copy(table_hbm.at[idx_v.at[j]],
                             rows_v.at[pl.ds(j * 128, 128)], sem)
            for j in range(4)
        ]
        for cp in cps:
            cp.wait()
        pltpu.sync_copy(rows_v, out_hbm.at[pl.ds(wid * rows_per_w, rows_per_w)])

    return k(table, gidx3)


def kernel(state, quantization_keys, in_proj_weight, in_proj_bias,
           embed0, embed1, embed2):
    bsz, T, Hsz = state.shape
    X = state.reshape(bsz, Hsz)
    wqt = in_proj_weight[:Hsz].T
    wkt = in_proj_weight[Hsz:2 * Hsz].T
    bq = in_proj_bias[:Hsz].reshape(1, Hsz)
    bk = in_proj_bias[Hsz:2 * Hsz].reshape(1, Hsz)
    keysp = jnp.zeros((8, Hsz), jnp.float32).at[:_NLV].set(
        quantization_keys.reshape(_NLV, Hsz))

    u = jax.random.uniform(jax.random.key(42), (1, bsz, _NLV),
                           minval=1e-10, maxval=1.0)
    g = -jnp.log(-jnp.log(u))[0]
    gn = jnp.concatenate(
        [g, jnp.full((bsz, 8 - _NLV), -jnp.inf, jnp.float32)], axis=1)

    pad = lambda e: jnp.pad(e, ((0, _NEP - _NE), (0, 0)))
    e0t = pad(embed0).T
    e1t = pad(embed1).T
    e2t = pad(embed2).T

    att8, gidx8, extra = _tc_call(X, keysp, wqt, wkt, bq, bk, gn,
                                  e0t, e1t, e2t)

    att = att8[:, :_NLV].reshape(1, bsz, _NLV)
    table = jnp.concatenate(
        [embed0.reshape(-1), embed1.reshape(-1),
         embed2.reshape(-1)]).reshape(-1, 64)
    gidx3 = gidx8[:, :4].reshape(32, 4, 128)
    out_rows = _sc_gather(table, gidx3)
    out = out_rows.reshape(bsz, T, Hsz)
    return out, extra[0, 0], att


# speedup vs baseline: 2.5469x; 1.0176x over previous
import functools

import jax
import jax.numpy as jnp
from jax import lax
from jax.experimental import pallas as pl
from jax.experimental.pallas import tpu as pltpu
from jax.experimental.pallas import tpu_sc as plsc

_N_FACTORS = (1, 2, 4)
_D = 256
_NE = 2730
_NEP = 2816
_H = 4
_DH = 64
_R = 256
_ALPHA = 0.01
_NLV = 3
_BSZ = 4096
_BIG = 1e30


def _tc_body(x_ref, keysp_ref, wqt_ref, wkt_ref, bq_ref, bk_ref, gn_ref,
             e0t_ref, e1t_ref, e2t_ref, att_ref, gidx_ref, extra_ref,
             he2_ref, et0_ref, et1_ref, et2_ref):
    i = pl.program_id(0)
    X = x_ref[...]
    lane8 = lax.broadcasted_iota(jnp.int32, (_R, 8), 1)

    lane_c = lax.broadcasted_iota(jnp.int32, (1, _NEP), 1)

    @pl.when(i == 0)
    def _():
        pad_pen = jnp.where(lane_c < _NE, jnp.float32(0.0), jnp.float32(_BIG))
        for r, ep_ref, et_ref in ((0, e0t_ref, et0_ref), (1, e1t_ref, et1_ref),
                                  (2, e2t_ref, et2_ref)):
            ET = jnp.transpose(ep_ref[...])
            et_ref[...] = ET
            he2_ref[r:r + 1, :] = (
                jnp.float32(0.5) * jnp.sum(ET * ET, axis=0, keepdims=True)
                + pad_pen)

    q = jnp.dot(X, wqt_ref[...], preferred_element_type=jnp.float32) + bq_ref[...]
    k = jnp.dot(keysp_ref[...], wkt_ref[...], preferred_element_type=jnp.float32) + bk_ref[...]
    scale = jnp.float32(_DH ** -0.5)
    att_soft = jnp.zeros((_R, 8), jnp.float32)
    for h in range(_H):
        qh = q[:, h * _DH:(h + 1) * _DH] * scale
        kh = k[:, h * _DH:(h + 1) * _DH]
        lg = lax.dot_general(qh, kh, (((1,), (1,)), ((), ())),
                             preferred_element_type=jnp.float32)
        lg = jnp.where(lane8 < _NLV, lg, -jnp.inf)
        mx = jnp.max(lg, axis=1, keepdims=True)
        ex = jnp.exp(lg - mx)
        att_soft = att_soft + ex / jnp.sum(ex, axis=1, keepdims=True)
    att_soft = att_soft * jnp.float32(1.0 / _H)

    yl = att_soft + gn_ref[...]
    mx = jnp.max(yl, axis=1, keepdims=True)
    ex = jnp.exp(yl - mx)
    y = ex / jnp.sum(ex, axis=1, keepdims=True)
    ymax = jnp.max(y, axis=1, keepdims=True)
    sel = jnp.min(jnp.where(y == ymax, lane8, 1 << 30), axis=1, keepdims=True)
    y_hard = jnp.where(lane8 == sel, jnp.float32(1.0), jnp.float32(0.0))
    att_out = (y_hard - y) + y
    att_ref[...] = att_out

    _W = 128
    _NCH = _NEP // _W
    lane_w = lax.broadcasted_iota(jnp.int32, (_R, _W), 1)

    _RH = 64
    lane_f = lax.broadcasted_iota(jnp.int32, (_RH, _W), 1).astype(jnp.float32)

    def level(Xs, ET, lv):
        x2 = jnp.sum(Xs * Xs, axis=1, keepdims=True)
        he2 = he2_ref[lv:lv + 1, :]
        m = jnp.dot(Xs, ET, preferred_element_type=jnp.float32)
        inds, smaxs = [], []
        for rb in range(_R // _RH):
            ms = m[rb * _RH:(rb + 1) * _RH]
            sacc = ms[:, :_W] - he2[:, :_W]
            iacc = lane_f
            for c in range(1, _NCH):
                s = (ms[:, c * _W:(c + 1) * _W]
                     - he2[:, c * _W:(c + 1) * _W])
                iacc = jnp.where(s > sacc, lane_f + jnp.float32(c * _W), iacc)
                sacc = jnp.maximum(sacc, s)
            mrow = jnp.max(sacc, axis=1, keepdims=True)
            indf = jnp.min(jnp.where(sacc == mrow, iacc, jnp.float32(1e9)),
                           axis=1, keepdims=True)
            inds.append(indf.astype(jnp.int32))
            smaxs.append(mrow)
        ind = jnp.concatenate(inds, axis=0)
        smax = jnp.concatenate(smaxs, axis=0)
        return ind, jnp.sum(x2 - 2.0 * smax)

    ind0, s0 = level(X, et0_ref[...], 0)
    E1 = et1_ref[...]
    ind1 = []
    s1 = jnp.float32(0.0)
    for sb in range(2):
        ind, s = level(X[:, sb * 128:(sb + 1) * 128], E1, 1)
        ind1.append(ind)
        s1 = s1 + s
    E2 = et2_ref[...]
    ind2 = []
    s2 = jnp.float32(0.0)
    for sb in range(4):
        ind, s = level(X[:, sb * 64:(sb + 1) * 64], E2, 2)
        ind2.append(ind)
        s2 = s2 + s

    g0 = ind0 * 4 + lane8
    g1 = (_NE * 4) + jnp.where(lane8 < 2, ind1[0], ind1[1]) * 2 + (lane8 & 1)
    c01 = jnp.where(lane8 == 0, ind2[0], ind2[1])
    c23 = jnp.where(lane8 == 2, ind2[2], ind2[3])
    g2 = (_NE * 6) + jnp.where(lane8 < 2, c01, c23)
    g = jnp.where(sel == 0, g0, jnp.where(sel == 1, g1, g2))
    gidx_ref[...] = jnp.where(lane8 < 4, g, 0)

    nvec = jnp.where(lane8 == 0, jnp.float32(1.0),
                     jnp.where(lane8 == 1, jnp.float32(2.0),
                               jnp.where(lane8 == 2, jnp.float32(4.0),
                                         jnp.float32(0.0))))
    pen_tile = jnp.sum(nvec * att_out)
    contrib = ((s0 + s1 + s2) * jnp.float32(1.0 / (_NLV * _BSZ * _D))
               + jnp.float32(_ALPHA / _BSZ) * pen_tile)

    @pl.when(i == 0)
    def _():
        extra_ref[...] = jnp.zeros((1, 1), jnp.float32)

    extra_ref[...] = extra_ref[...] + jnp.reshape(contrib, (1, 1))


def _tc_call(X, keysp, wqt, wkt, bq, bk, gn, e0t, e1t, e2t):
    n_tiles = _BSZ // _R
    full = lambda shape: pl.BlockSpec(shape, lambda i: (0, 0))
    return pl.pallas_call(
        _tc_body,
        grid=(n_tiles,),
        in_specs=[
            pl.BlockSpec((_R, _D), lambda i: (i, 0)),
            full((8, _D)),
            full((_D, _D)),
            full((_D, _D)),
            full((1, _D)),
            full((1, _D)),
            pl.BlockSpec((_R, 8), lambda i: (i, 0)),
            full((_NEP, _D)),
            full((_NEP, 128)),
            full((_NEP, 64)),
        ],
        out_specs=[
            pl.BlockSpec((_R, 8), lambda i: (i, 0)),
            pl.BlockSpec((_R, 8), lambda i: (i, 0)),
            pl.BlockSpec((1, 1), lambda i: (0, 0)),
        ],
        out_shape=[
            jax.ShapeDtypeStruct((_BSZ, 8), jnp.float32),
            jax.ShapeDtypeStruct((_BSZ, 8), jnp.int32),
            jax.ShapeDtypeStruct((1, 1), jnp.float32),
        ],
        scratch_shapes=[pltpu.VMEM((8, _NEP), jnp.float32),
                        pltpu.VMEM((_D, _NEP), jnp.float32),
                        pltpu.VMEM((128, _NEP), jnp.float32),
                        pltpu.VMEM((64, _NEP), jnp.float32)],
    )(X, keysp, wqt, wkt, bq, bk, gn, e0t, e1t, e2t)


def _sc_gather(table, gidx3):
    info = plsc.get_sparse_core_info()
    nc, ns = info.num_cores, info.num_subcores
    nw = nc * ns
    rows_per_w = 16384 // nw
    mesh = plsc.VectorSubcoreMesh(core_axis_name="c", subcore_axis_name="s")

    @functools.partial(
        pl.kernel,
        mesh=mesh,
        compiler_params=pltpu.CompilerParams(use_tc_tiling_on_sc=False),
        out_type=jax.ShapeDtypeStruct((16384, 64), jnp.float32),
        scratch_types=[
            pltpu.VMEM((4, 128), jnp.int32),
            pltpu.VMEM((rows_per_w, 64), jnp.float32),
            pltpu.SemaphoreType.DMA,
        ],
    )
    def k(table_hbm, idx_hbm, out_hbm, idx_v, rows_v, sem):
        wid = lax.axis_index("s") * nc + lax.axis_index("c")
        pltpu.sync_copy(idx_hbm.at[wid], idx_v)
        cps = [
            pltpu.async_copy(table_hbm.at[idx_v.at[j]],
                             rows_v.at[pl.ds(j * 128, 128)], sem)
            for j in range(4)
        ]
        for cp in cps:
            cp.wait()
        pltpu.sync_copy(rows_v, out_hbm.at[pl.ds(wid * rows_per_w, rows_per_w)])

    return k(table, gidx3)


def kernel(state, quantization_keys, in_proj_weight, in_proj_bias,
           embed0, embed1, embed2):
    bsz, T, Hsz = state.shape
    X = state.reshape(bsz, Hsz)
    wqt = in_proj_weight[:Hsz].T
    wkt = in_proj_weight[Hsz:2 * Hsz].T
    bq = in_proj_bias[:Hsz].reshape(1, Hsz)
    bk = in_proj_bias[Hsz:2 * Hsz].reshape(1, Hsz)
    keysp = jnp.zeros((8, Hsz), jnp.float32).at[:_NLV].set(
        quantization_keys.reshape(_NLV, Hsz))

    u = jax.random.uniform(jax.random.key(42), (1, bsz, _NLV),
                           minval=1e-10, maxval=1.0)
    g = -jnp.log(-jnp.log(u))[0]
    gn = jnp.concatenate(
        [g, jnp.full((bsz, 8 - _NLV), -jnp.inf, jnp.float32)], axis=1)

    pad = lambda e: jnp.pad(e, ((0, _NEP - _NE), (0, 0)))
    e0t = pad(embed0)
    e1t = pad(embed1)
    e2t = pad(embed2)

    att8, gidx8, extra = _tc_call(X, keysp, wqt, wkt, bq, bk, gn,
                                  e0t, e1t, e2t)

    att = att8[:, :_NLV].reshape(1, bsz, _NLV)
    table = jnp.concatenate(
        [embed0.reshape(-1), embed1.reshape(-1),
         embed2.reshape(-1)]).reshape(-1, 64)
    gidx3 = gidx8[:, :4].reshape(32, 4, 128)
    out_rows = _sc_gather(table, gidx3)
    out = out_rows.reshape(bsz, T, Hsz)
    return out, extra[0, 0], att


# speedup vs baseline: 2.7505x; 1.0800x over previous
import functools

import numpy as np

import jax
import jax.numpy as jnp
from jax import lax
from jax.experimental import pallas as pl
from jax.experimental.pallas import tpu as pltpu
from jax.experimental.pallas import tpu_sc as plsc

_N_FACTORS = (1, 2, 4)
_D = 256
_NE = 2730
_NEP = 2816
_H = 4
_DH = 64
_R = 256
_ALPHA = 0.01
_NLV = 3
_BSZ = 4096
_BIG = 1e30

def _gumbel_noise():
    u = jax.random.uniform(jax.random.key(42), (1, _BSZ, _NLV),
                           minval=1e-10, maxval=1.0)
    return np.asarray(-jnp.log(-jnp.log(u))[0])


_GN = _gumbel_noise()


def _tc_body(x_ref, keys_ref, wqt_ref, wkt_ref, bq_ref, bk_ref, gn_ref,
             e0t_ref, e1t_ref, e2t_ref, att_ref, gidx_ref, extra_ref,
             he2_ref, et0_ref, et1_ref, et2_ref):
    i = pl.program_id(0)
    X = x_ref[...]
    lane8 = lax.broadcasted_iota(jnp.int32, (_R, 8), 1)

    lane_c = lax.broadcasted_iota(jnp.int32, (1, _NEP), 1)

    @pl.when(i == 0)
    def _():
        pad_pen = jnp.where(lane_c < _NE, jnp.float32(0.0), jnp.float32(_BIG))
        for r, ep_ref, et_ref in ((0, e0t_ref, et0_ref), (1, e1t_ref, et1_ref),
                                  (2, e2t_ref, et2_ref)):
            ET = jnp.transpose(ep_ref[...])
            et_ref[...] = ET
            he2_ref[r:r + 1, :] = (
                jnp.float32(0.5) * jnp.sum(ET * ET, axis=0, keepdims=True)
                + pad_pen)

    lane3 = lax.broadcasted_iota(jnp.int32, (_R, _NLV), 1)
    q = jnp.dot(X, wqt_ref[...], preferred_element_type=jnp.float32) + bq_ref[...]
    k = jnp.dot(keys_ref[...], wkt_ref[...], preferred_element_type=jnp.float32) + bk_ref[...]
    scale = jnp.float32(_DH ** -0.5)
    att_soft = jnp.zeros((_R, _NLV), jnp.float32)
    for h in range(_H):
        qh = q[:, h * _DH:(h + 1) * _DH] * scale
        kh = k[:, h * _DH:(h + 1) * _DH]
        lg = lax.dot_general(qh, kh, (((1,), (1,)), ((), ())),
                             preferred_element_type=jnp.float32)
        mx = jnp.max(lg, axis=1, keepdims=True)
        ex = jnp.exp(lg - mx)
        att_soft = att_soft + ex / jnp.sum(ex, axis=1, keepdims=True)
    att_soft = att_soft * jnp.float32(1.0 / _H)

    yl = att_soft + gn_ref[...]
    mx = jnp.max(yl, axis=1, keepdims=True)
    ex = jnp.exp(yl - mx)
    y = ex / jnp.sum(ex, axis=1, keepdims=True)
    ymax = jnp.max(y, axis=1, keepdims=True)
    sel = jnp.min(jnp.where(y == ymax, lane3, 1 << 30), axis=1, keepdims=True)
    y_hard = jnp.where(lane3 == sel, jnp.float32(1.0), jnp.float32(0.0))
    att_out = (y_hard - y) + y
    att_ref[...] = att_out

    _W = 128
    _NCH = _NEP // _W
    lane_w = lax.broadcasted_iota(jnp.int32, (_R, _W), 1)

    _RH = 64
    lane_f = lax.broadcasted_iota(jnp.int32, (_RH, _W), 1).astype(jnp.float32)

    def level(Xs, ET, lv):
        x2 = jnp.sum(Xs * Xs, axis=1, keepdims=True)
        he2 = he2_ref[lv:lv + 1, :]
        m = jnp.dot(Xs, ET, preferred_element_type=jnp.float32)
        inds, smaxs = [], []
        for rb in range(_R // _RH):
            ms = m[rb * _RH:(rb + 1) * _RH]
            sacc = ms[:, :_W] - he2[:, :_W]
            iacc = lane_f
            for c in range(1, _NCH):
                s = (ms[:, c * _W:(c + 1) * _W]
                     - he2[:, c * _W:(c + 1) * _W])
                iacc = jnp.where(s > sacc, lane_f + jnp.float32(c * _W), iacc)
                sacc = jnp.maximum(sacc, s)
            mrow = jnp.max(sacc, axis=1, keepdims=True)
            indf = jnp.min(jnp.where(sacc == mrow, iacc, jnp.float32(1e9)),
                           axis=1, keepdims=True)
            inds.append(indf.astype(jnp.int32))
            smaxs.append(mrow)
        ind = jnp.concatenate(inds, axis=0)
        smax = jnp.concatenate(smaxs, axis=0)
        return ind, jnp.sum(x2 - 2.0 * smax)

    ind0, s0 = level(X, et0_ref[...], 0)
    E1 = et1_ref[...]
    ind1 = []
    s1 = jnp.float32(0.0)
    for sb in range(2):
        ind, s = level(X[:, sb * 128:(sb + 1) * 128], E1, 1)
        ind1.append(ind)
        s1 = s1 + s
    E2 = et2_ref[...]
    ind2 = []
    s2 = jnp.float32(0.0)
    for sb in range(4):
        ind, s = level(X[:, sb * 64:(sb + 1) * 64], E2, 2)
        ind2.append(ind)
        s2 = s2 + s

    g0 = ind0 * 4 + lane8
    g1 = (_NE * 4) + jnp.where(lane8 < 2, ind1[0], ind1[1]) * 2 + (lane8 & 1)
    c01 = jnp.where(lane8 == 0, ind2[0], ind2[1])
    c23 = jnp.where(lane8 == 2, ind2[2], ind2[3])
    g2 = (_NE * 6) + jnp.where(lane8 < 2, c01, c23)
    g = jnp.where(sel == 0, g0, jnp.where(sel == 1, g1, g2))
    gidx_ref[...] = jnp.where(lane8 < 4, g, 0)

    nvec3 = jnp.where(lane3 == 0, jnp.float32(1.0),
                      jnp.where(lane3 == 1, jnp.float32(2.0),
                                jnp.float32(4.0)))
    pen_tile = jnp.sum(nvec3 * att_out)
    contrib = ((s0 + s1 + s2) * jnp.float32(1.0 / (_NLV * _BSZ * _D))
               + jnp.float32(_ALPHA / _BSZ) * pen_tile)

    @pl.when(i == 0)
    def _():
        extra_ref[...] = jnp.zeros((1, 1), jnp.float32)

    extra_ref[...] = extra_ref[...] + jnp.reshape(contrib, (1, 1))


def _tc_call(X, keysp, wqt, wkt, bq, bk, gn, e0t, e1t, e2t):
    n_tiles = _BSZ // _R
    full = lambda shape: pl.BlockSpec(shape, lambda i: (0, 0))
    return pl.pallas_call(
        _tc_body,
        grid=(n_tiles,),
        in_specs=[
            pl.BlockSpec((_R, _D), lambda i: (i, 0)),
            full((_NLV, _D)),
            full((_D, _D)),
            full((_D, _D)),
            full((1, _D)),
            full((1, _D)),
            pl.BlockSpec((_R, _NLV), lambda i: (i, 0)),
            full((_NEP, _D)),
            full((_NEP, 128)),
            full((_NEP, 64)),
        ],
        out_specs=[
            pl.BlockSpec((_R, _NLV), lambda i: (i, 0)),
            pl.BlockSpec((_R, 8), lambda i: (i, 0)),
            pl.BlockSpec((1, 1), lambda i: (0, 0)),
        ],
        out_shape=[
            jax.ShapeDtypeStruct((_BSZ, _NLV), jnp.float32),
            jax.ShapeDtypeStruct((_BSZ, 8), jnp.int32),
            jax.ShapeDtypeStruct((1, 1), jnp.float32),
        ],
        scratch_shapes=[pltpu.VMEM((8, _NEP), jnp.float32),
                        pltpu.VMEM((_D, _NEP), jnp.float32),
                        pltpu.VMEM((128, _NEP), jnp.float32),
                        pltpu.VMEM((64, _NEP), jnp.float32)],
    )(X, keysp, wqt, wkt, bq, bk, gn, e0t, e1t, e2t)


def _sc_gather(table, gidx3):
    info = plsc.get_sparse_core_info()
    nc, ns = info.num_cores, info.num_subcores
    nw = nc * ns
    rows_per_w = 16384 // nw
    mesh = plsc.VectorSubcoreMesh(core_axis_name="c", subcore_axis_name="s")

    @functools.partial(
        pl.kernel,
        mesh=mesh,
        compiler_params=pltpu.CompilerParams(use_tc_tiling_on_sc=False),
        out_type=jax.ShapeDtypeStruct((16384, 64), jnp.float32),
        scratch_types=[
            pltpu.VMEM((4, 128), jnp.int32),
            pltpu.VMEM((rows_per_w, 64), jnp.float32),
            pltpu.SemaphoreType.DMA,
        ],
    )
    def k(table_hbm, idx_hbm, out_hbm, idx_v, rows_v, sem):
        wid = lax.axis_index("s") * nc + lax.axis_index("c")
        pltpu.sync_copy(idx_hbm.at[wid], idx_v)
        cps = [
            pltpu.async_copy(table_hbm.at[idx_v.at[j]],
                             rows_v.at[pl.ds(j * 128, 128)], sem)
            for j in range(4)
        ]
        for cp in cps:
            cp.wait()
        pltpu.sync_copy(rows_v, out_hbm.at[pl.ds(wid * rows_per_w, rows_per_w)])

    return k(table, gidx3)


def kernel(state, quantization_keys, in_proj_weight, in_proj_bias,
           embed0, embed1, embed2):
    bsz, T, Hsz = state.shape
    X = state.reshape(bsz, Hsz)
    wqt = in_proj_weight[:Hsz].T
    wkt = in_proj_weight[Hsz:2 * Hsz].T
    bq = in_proj_bias[:Hsz].reshape(1, Hsz)
    bk = in_proj_bias[Hsz:2 * Hsz].reshape(1, Hsz)
    keys3 = quantization_keys.reshape(_NLV, Hsz)
    gn = _GN

    pad = lambda e: jnp.pad(e, ((0, _NEP - _NE), (0, 0)))
    e0t = pad(embed0)
    e1t = pad(embed1)
    e2t = pad(embed2)

    att3, gidx8, extra = _tc_call(X, keys3, wqt, wkt, bq, bk, gn,
                                  e0t, e1t, e2t)

    att = att3.reshape(1, bsz, _NLV)
    table = jnp.concatenate(
        [embed0.reshape(-1), embed1.reshape(-1),
         embed2.reshape(-1)]).reshape(-1, 64)
    gidx3 = gidx8[:, :4].reshape(32, 4, 128)
    out_rows = _sc_gather(table, gidx3)
    out = out_rows.reshape(bsz, T, Hsz)
    return out, extra[0, 0], att


# speedup vs baseline: 2.7748x; 1.0088x over previous
import functools

import numpy as np

import jax
import jax.numpy as jnp
from jax import lax
from jax.experimental import pallas as pl
from jax.experimental.pallas import tpu as pltpu
from jax.experimental.pallas import tpu_sc as plsc

_N_FACTORS = (1, 2, 4)
_D = 256
_NE = 2730
_NEP = 2816
_H = 4
_DH = 64
_R = 256
_ALPHA = 0.01
_NLV = 3
_BSZ = 4096
_BIG = 1e30

def _gumbel_noise_graph():
    u = jax.random.uniform(jax.random.key(42), (1, _BSZ, _NLV),
                           minval=1e-10, maxval=1.0)
    return -jnp.log(-jnp.log(u))[0]


try:
    _GN = np.asarray(jax.jit(_gumbel_noise_graph, backend="cpu")())
except Exception:
    _GN = None


def _tc_body(x_ref, keys_ref, w_ref, bq_ref, bk_ref, gn_ref,
             e0t_ref, e1t_ref, e2t_ref, att_ref, gidx_ref, extra_ref,
             he2_ref, et0_ref, et1_ref, et2_ref, wqt_ref, wkt_ref):
    i = pl.program_id(0)
    X = x_ref[...]
    lane8 = lax.broadcasted_iota(jnp.int32, (_R, 8), 1)

    lane_c = lax.broadcasted_iota(jnp.int32, (1, _NEP), 1)

    @pl.when(i == 0)
    def _():
        pad_pen = jnp.where(lane_c < _NE, jnp.float32(0.0), jnp.float32(_BIG))
        for r, ep_ref, et_ref in ((0, e0t_ref, et0_ref), (1, e1t_ref, et1_ref),
                                  (2, e2t_ref, et2_ref)):
            ET = jnp.transpose(ep_ref[...])
            et_ref[...] = ET
            he2_ref[r:r + 1, :] = (
                jnp.float32(0.5) * jnp.sum(ET * ET, axis=0, keepdims=True)
                + pad_pen)
        wqt_ref[...] = jnp.transpose(w_ref[0:_D, :])
        wkt_ref[...] = jnp.transpose(w_ref[_D:2 * _D, :])

    lane3 = lax.broadcasted_iota(jnp.int32, (_R, _NLV), 1)
    q = jnp.dot(X, wqt_ref[...], preferred_element_type=jnp.float32) + bq_ref[...]
    k = jnp.dot(keys_ref[...], wkt_ref[...],
                preferred_element_type=jnp.float32) + bk_ref[...]
    scale = jnp.float32(_DH ** -0.5)
    att_soft = jnp.zeros((_R, _NLV), jnp.float32)
    for h in range(_H):
        qh = q[:, h * _DH:(h + 1) * _DH] * scale
        kh = k[:, h * _DH:(h + 1) * _DH]
        lg = lax.dot_general(qh, kh, (((1,), (1,)), ((), ())),
                             preferred_element_type=jnp.float32)
        mx = jnp.max(lg, axis=1, keepdims=True)
        ex = jnp.exp(lg - mx)
        att_soft = att_soft + ex / jnp.sum(ex, axis=1, keepdims=True)
    att_soft = att_soft * jnp.float32(1.0 / _H)

    yl = att_soft + gn_ref[...]
    mx = jnp.max(yl, axis=1, keepdims=True)
    ex = jnp.exp(yl - mx)
    y = ex / jnp.sum(ex, axis=1, keepdims=True)
    ymax = jnp.max(y, axis=1, keepdims=True)
    sel = jnp.min(jnp.where(y == ymax, lane3, 1 << 30), axis=1, keepdims=True)
    y_hard = jnp.where(lane3 == sel, jnp.float32(1.0), jnp.float32(0.0))
    att_out = (y_hard - y) + y
    att_ref[...] = att_out

    _W = 128
    _NCH = _NEP // _W
    lane_w = lax.broadcasted_iota(jnp.int32, (_R, _W), 1)

    _RH = 64
    lane_f = lax.broadcasted_iota(jnp.int32, (_RH, _W), 1).astype(jnp.float32)

    def level(Xs, ET, lv):
        x2 = jnp.sum(Xs * Xs, axis=1, keepdims=True)
        he2 = he2_ref[lv:lv + 1, :]
        m = jnp.dot(Xs, ET, preferred_element_type=jnp.float32)
        inds, smaxs = [], []
        for rb in range(_R // _RH):
            ms = m[rb * _RH:(rb + 1) * _RH]
            sacc = ms[:, :_W] - he2[:, :_W]
            iacc = lane_f
            for c in range(1, _NCH):
                s = (ms[:, c * _W:(c + 1) * _W]
                     - he2[:, c * _W:(c + 1) * _W])
                iacc = jnp.where(s > sacc, lane_f + jnp.float32(c * _W), iacc)
                sacc = jnp.maximum(sacc, s)
            mrow = jnp.max(sacc, axis=1, keepdims=True)
            indf = jnp.min(jnp.where(sacc == mrow, iacc, jnp.float32(1e9)),
                           axis=1, keepdims=True)
            inds.append(indf.astype(jnp.int32))
            smaxs.append(mrow)
        ind = jnp.concatenate(inds, axis=0)
        smax = jnp.concatenate(smaxs, axis=0)
        return ind, jnp.sum(x2 - 2.0 * smax)

    ind0, s0 = level(X, et0_ref[...], 0)
    E1 = et1_ref[...]
    ind1 = []
    s1 = jnp.float32(0.0)
    for sb in range(2):
        ind, s = level(X[:, sb * 128:(sb + 1) * 128], E1, 1)
        ind1.append(ind)
        s1 = s1 + s
    E2 = et2_ref[...]
    ind2 = []
    s2 = jnp.float32(0.0)
    for sb in range(4):
        ind, s = level(X[:, sb * 64:(sb + 1) * 64], E2, 2)
        ind2.append(ind)
        s2 = s2 + s

    g0 = ind0 * 4 + lane8
    g1 = (_NE * 4) + jnp.where(lane8 < 2, ind1[0], ind1[1]) * 2 + (lane8 & 1)
    c01 = jnp.where(lane8 == 0, ind2[0], ind2[1])
    c23 = jnp.where(lane8 == 2, ind2[2], ind2[3])
    g2 = (_NE * 6) + jnp.where(lane8 < 2, c01, c23)
    g = jnp.where(sel == 0, g0, jnp.where(sel == 1, g1, g2))
    gidx_ref[...] = jnp.where(lane8 < 4, g, 0)

    nvec3 = jnp.where(lane3 == 0, jnp.float32(1.0),
                      jnp.where(lane3 == 1, jnp.float32(2.0),
                                jnp.float32(4.0)))
    pen_tile = jnp.sum(nvec3 * att_out)
    contrib = ((s0 + s1 + s2) * jnp.float32(1.0 / (_NLV * _BSZ * _D))
               + jnp.float32(_ALPHA / _BSZ) * pen_tile)

    @pl.when(i == 0)
    def _():
        extra_ref[...] = jnp.zeros((1, 1), jnp.float32)

    extra_ref[...] = extra_ref[...] + jnp.reshape(contrib, (1, 1))


def _tc_call(X3, keys3, w, bq, bk, gn, e0t, e1t, e2t):
    n_tiles = _BSZ // _R
    full = lambda shape: pl.BlockSpec(shape, lambda i: (0, 0))
    return pl.pallas_call(
        _tc_body,
        grid=(n_tiles,),
        in_specs=[
            pl.BlockSpec((_R, _D), lambda i: (i, 0)),
            full((_NLV, _D)),
            pl.BlockSpec((3 * _D, _D), lambda i: (0, 0)),
            full((1, _D)),
            full((1, _D)),
            pl.BlockSpec((_R, _NLV), lambda i: (i, 0)),
            full((_NEP, _D)),
            full((_NEP, 128)),
            full((_NEP, 64)),
        ],
        out_specs=[
            pl.BlockSpec((_R, _NLV), lambda i: (i, 0)),
            pl.BlockSpec((_R, 8), lambda i: (i, 0)),
            pl.BlockSpec((1, 1), lambda i: (0, 0)),
        ],
        out_shape=[
            jax.ShapeDtypeStruct((_BSZ, _NLV), jnp.float32),
            jax.ShapeDtypeStruct((_BSZ, 8), jnp.int32),
            jax.ShapeDtypeStruct((1, 1), jnp.float32),
        ],
        scratch_shapes=[pltpu.VMEM((8, _NEP), jnp.float32),
                        pltpu.VMEM((_D, _NEP), jnp.float32),
                        pltpu.VMEM((128, _NEP), jnp.float32),
                        pltpu.VMEM((64, _NEP), jnp.float32),
                        pltpu.VMEM((_D, _D), jnp.float32),
                        pltpu.VMEM((_D, _D), jnp.float32)],
    )(X3, keys3, w, bq, bk, gn, e0t, e1t, e2t)


def _sc_gather(table, gidx3):
    info = plsc.get_sparse_core_info()
    nc, ns = info.num_cores, info.num_subcores
    nw = nc * ns
    rows_per_w = 16384 // nw
    mesh = plsc.VectorSubcoreMesh(core_axis_name="c", subcore_axis_name="s")

    @functools.partial(
        pl.kernel,
        mesh=mesh,
        compiler_params=pltpu.CompilerParams(use_tc_tiling_on_sc=False),
        out_type=jax.ShapeDtypeStruct((16384, 64), jnp.float32),
        scratch_types=[
            pltpu.VMEM((4, 128), jnp.int32),
            pltpu.VMEM((rows_per_w, 64), jnp.float32),
            pltpu.SemaphoreType.DMA,
        ],
    )
    def k(table_hbm, idx_hbm, out_hbm, idx_v, rows_v, sem):
        wid = lax.axis_index("s") * nc + lax.axis_index("c")
        pltpu.sync_copy(idx_hbm.at[wid], idx_v)
        cps = [
            pltpu.async_copy(table_hbm.at[idx_v.at[j]],
                             rows_v.at[pl.ds(j * 128, 128)], sem)
            for j in range(4)
        ]
        for cp in cps:
            cp.wait()
        pltpu.sync_copy(rows_v, out_hbm.at[pl.ds(wid * rows_per_w, rows_per_w)])

    return k(table, gidx3)


def kernel(state, quantization_keys, in_proj_weight, in_proj_bias,
           embed0, embed1, embed2):
    bsz, T, Hsz = state.shape
    bq = in_proj_bias[:Hsz].reshape(1, Hsz)
    bk = in_proj_bias[Hsz:2 * Hsz].reshape(1, Hsz)
    keys3 = quantization_keys.reshape(_NLV, Hsz)
    gn = _GN if _GN is not None else _gumbel_noise_graph()

    pad = lambda e: jnp.pad(e, ((0, _NEP - _NE), (0, 0)))
    e0t = pad(embed0)
    e1t = pad(embed1)
    e2t = pad(embed2)

    att3, gidx8, extra = _tc_call(state.reshape(bsz, Hsz), keys3,
                                  in_proj_weight, bq, bk, gn, e0t, e1t, e2t)

    att = att3.reshape(1, bsz, _NLV)
    table = jnp.concatenate(
        [embed0.reshape(-1), embed1.reshape(-1),
         embed2.reshape(-1)]).reshape(-1, 64)
    gidx3 = gidx8[:, :4].reshape(32, 4, 128)
    out_rows = _sc_gather(table, gidx3)
    out = out_rows.reshape(bsz, T, Hsz)
    return out, extra[0, 0], att


# speedup vs baseline: 2.8244x; 1.0179x over previous
import functools

import numpy as np

import jax
import jax.numpy as jnp
from jax import lax
from jax.experimental import pallas as pl
from jax.experimental.pallas import tpu as pltpu
from jax.experimental.pallas import tpu_sc as plsc

_N_FACTORS = (1, 2, 4)
_D = 256
_NE = 2730
_NEP = 2816
_H = 4
_DH = 64
_R = 512
_ALPHA = 0.01
_NLV = 3
_BSZ = 4096
_BIG = 1e30

def _gumbel_noise_graph():
    u = jax.random.uniform(jax.random.key(42), (1, _BSZ, _NLV),
                           minval=1e-10, maxval=1.0)
    return -jnp.log(-jnp.log(u))[0]


try:
    _GN = np.asarray(jax.jit(_gumbel_noise_graph, backend="cpu")())
except Exception:
    _GN = None


def _tc_body(x_ref, keys_ref, w_ref, bq_ref, bk_ref, gn_ref,
             e0t_ref, e1t_ref, e2t_ref, att_ref, gidx_ref, extra_ref,
             he2_ref, et0_ref, et1_ref, et2_ref, wqt_ref, wkt_ref):
    i = pl.program_id(0)
    X = x_ref[...]
    lane8 = lax.broadcasted_iota(jnp.int32, (_R, 8), 1)

    lane_c = lax.broadcasted_iota(jnp.int32, (1, _NEP), 1)

    @pl.when(i == 0)
    def _():
        pad_pen = jnp.where(lane_c < _NE, jnp.float32(0.0), jnp.float32(_BIG))
        for r, ep_ref, et_ref in ((0, e0t_ref, et0_ref), (1, e1t_ref, et1_ref),
                                  (2, e2t_ref, et2_ref)):
            ET = jnp.transpose(ep_ref[...])
            et_ref[...] = ET
            he2_ref[r:r + 1, :] = (
                jnp.float32(0.5) * jnp.sum(ET * ET, axis=0, keepdims=True)
                + pad_pen)
        wqt_ref[...] = jnp.transpose(w_ref[0:_D, :])
        wkt_ref[...] = jnp.transpose(w_ref[_D:2 * _D, :])

    lane3 = lax.broadcasted_iota(jnp.int32, (_R, _NLV), 1)
    q = jnp.dot(X, wqt_ref[...], preferred_element_type=jnp.float32) + bq_ref[...]
    k = jnp.dot(keys_ref[...], wkt_ref[...],
                preferred_element_type=jnp.float32) + bk_ref[...]
    scale = jnp.float32(_DH ** -0.5)
    att_soft = jnp.zeros((_R, _NLV), jnp.float32)
    for h in range(_H):
        qh = q[:, h * _DH:(h + 1) * _DH] * scale
        kh = k[:, h * _DH:(h + 1) * _DH]
        lg = lax.dot_general(qh, kh, (((1,), (1,)), ((), ())),
                             preferred_element_type=jnp.float32)
        mx = jnp.max(lg, axis=1, keepdims=True)
        ex = jnp.exp(lg - mx)
        att_soft = att_soft + ex / jnp.sum(ex, axis=1, keepdims=True)
    att_soft = att_soft * jnp.float32(1.0 / _H)

    yl = att_soft + gn_ref[...]
    mx = jnp.max(yl, axis=1, keepdims=True)
    ex = jnp.exp(yl - mx)
    y = ex / jnp.sum(ex, axis=1, keepdims=True)
    ymax = jnp.max(y, axis=1, keepdims=True)
    sel = jnp.min(jnp.where(y == ymax, lane3, 1 << 30), axis=1, keepdims=True)
    y_hard = jnp.where(lane3 == sel, jnp.float32(1.0), jnp.float32(0.0))
    att_out = (y_hard - y) + y
    att_ref[...] = att_out

    _W = 128
    _NCH = _NEP // _W
    lane_w = lax.broadcasted_iota(jnp.int32, (_R, _W), 1)

    _RH = 64
    lane_f = lax.broadcasted_iota(jnp.int32, (_RH, _W), 1).astype(jnp.float32)

    def level(Xs, ET, lv):
        x2 = jnp.sum(Xs * Xs, axis=1, keepdims=True)
        he2 = he2_ref[lv:lv + 1, :]
        m = jnp.dot(Xs, ET, preferred_element_type=jnp.float32)
        inds, smaxs = [], []
        for rb in range(_R // _RH):
            ms = m[rb * _RH:(rb + 1) * _RH]
            sacc = ms[:, :_W] - he2[:, :_W]
            iacc = lane_f
            for c in range(1, _NCH):
                s = (ms[:, c * _W:(c + 1) * _W]
                     - he2[:, c * _W:(c + 1) * _W])
                iacc = jnp.where(s > sacc, lane_f + jnp.float32(c * _W), iacc)
                sacc = jnp.maximum(sacc, s)
            mrow = jnp.max(sacc, axis=1, keepdims=True)
            indf = jnp.min(jnp.where(sacc == mrow, iacc, jnp.float32(1e9)),
                           axis=1, keepdims=True)
            inds.append(indf.astype(jnp.int32))
            smaxs.append(mrow)
        ind = jnp.concatenate(inds, axis=0)
        smax = jnp.concatenate(smaxs, axis=0)
        return ind, jnp.sum(x2 - 2.0 * smax)

    ind0, s0 = level(X, et0_ref[...], 0)
    E1 = et1_ref[...]
    ind1 = []
    s1 = jnp.float32(0.0)
    for sb in range(2):
        ind, s = level(X[:, sb * 128:(sb + 1) * 128], E1, 1)
        ind1.append(ind)
        s1 = s1 + s
    E2 = et2_ref[...]
    ind2 = []
    s2 = jnp.float32(0.0)
    for sb in range(4):
        ind, s = level(X[:, sb * 64:(sb + 1) * 64], E2, 2)
        ind2.append(ind)
        s2 = s2 + s

    g0 = ind0 * 4 + lane8
    g1 = (_NE * 4) + jnp.where(lane8 < 2, ind1[0], ind1[1]) * 2 + (lane8 & 1)
    c01 = jnp.where(lane8 == 0, ind2[0], ind2[1])
    c23 = jnp.where(lane8 == 2, ind2[2], ind2[3])
    g2 = (_NE * 6) + jnp.where(lane8 < 2, c01, c23)
    g = jnp.where(sel == 0, g0, jnp.where(sel == 1, g1, g2))
    gidx_ref[...] = jnp.where(lane8 < 4, g, 0)

    nvec3 = jnp.where(lane3 == 0, jnp.float32(1.0),
                      jnp.where(lane3 == 1, jnp.float32(2.0),
                                jnp.float32(4.0)))
    pen_tile = jnp.sum(nvec3 * att_out)
    contrib = ((s0 + s1 + s2) * jnp.float32(1.0 / (_NLV * _BSZ * _D))
               + jnp.float32(_ALPHA / _BSZ) * pen_tile)

    @pl.when(i == 0)
    def _():
        extra_ref[...] = jnp.zeros((1, 1), jnp.float32)

    extra_ref[...] = extra_ref[...] + jnp.reshape(contrib, (1, 1))


def _tc_call(X3, keys3, w, bq, bk, gn, e0t, e1t, e2t):
    n_tiles = _BSZ // _R
    full = lambda shape: pl.BlockSpec(shape, lambda i: (0, 0))
    return pl.pallas_call(
        _tc_body,
        grid=(n_tiles,),
        in_specs=[
            pl.BlockSpec((_R, _D), lambda i: (i, 0)),
            full((_NLV, _D)),
            pl.BlockSpec((3 * _D, _D), lambda i: (0, 0)),
            full((1, _D)),
            full((1, _D)),
            pl.BlockSpec((_R, _NLV), lambda i: (i, 0)),
            full((_NEP, _D)),
            full((_NEP, 128)),
            full((_NEP, 64)),
        ],
        out_specs=[
            pl.BlockSpec((_R, _NLV), lambda i: (i, 0)),
            pl.BlockSpec((_R, 8), lambda i: (i, 0)),
            pl.BlockSpec((1, 1), lambda i: (0, 0)),
        ],
        out_shape=[
            jax.ShapeDtypeStruct((_BSZ, _NLV), jnp.float32),
            jax.ShapeDtypeStruct((_BSZ, 8), jnp.int32),
            jax.ShapeDtypeStruct((1, 1), jnp.float32),
        ],
        scratch_shapes=[pltpu.VMEM((8, _NEP), jnp.float32),
                        pltpu.VMEM((_D, _NEP), jnp.float32),
                        pltpu.VMEM((128, _NEP), jnp.float32),
                        pltpu.VMEM((64, _NEP), jnp.float32),
                        pltpu.VMEM((_D, _D), jnp.float32),
                        pltpu.VMEM((_D, _D), jnp.float32)],
    )(X3, keys3, w, bq, bk, gn, e0t, e1t, e2t)


def _sc_gather(table, gidx3):
    info = plsc.get_sparse_core_info()
    nc, ns = info.num_cores, info.num_subcores
    nw = nc * ns
    rows_per_w = 16384 // nw
    mesh = plsc.VectorSubcoreMesh(core_axis_name="c", subcore_axis_name="s")

    @functools.partial(
        pl.kernel,
        mesh=mesh,
        compiler_params=pltpu.CompilerParams(use_tc_tiling_on_sc=False),
        out_type=jax.ShapeDtypeStruct((16384, 64), jnp.float32),
        scratch_types=[
            pltpu.VMEM((4, 128), jnp.int32),
            pltpu.VMEM((rows_per_w, 64), jnp.float32),
            pltpu.SemaphoreType.DMA,
        ],
    )
    def k(table_hbm, idx_hbm, out_hbm, idx_v, rows_v, sem):
        wid = lax.axis_index("s") * nc + lax.axis_index("c")
        pltpu.sync_copy(idx_hbm.at[wid], idx_v)
        cps = [
            pltpu.async_copy(table_hbm.at[idx_v.at[j]],
                             rows_v.at[pl.ds(j * 128, 128)], sem)
            for j in range(4)
        ]
        for cp in cps:
            cp.wait()
        pltpu.sync_copy(rows_v, out_hbm.at[pl.ds(wid * rows_per_w, rows_per_w)])

    return k(table, gidx3)


def kernel(state, quantization_keys, in_proj_weight, in_proj_bias,
           embed0, embed1, embed2):
    bsz, T, Hsz = state.shape
    bq = in_proj_bias[:Hsz].reshape(1, Hsz)
    bk = in_proj_bias[Hsz:2 * Hsz].reshape(1, Hsz)
    keys3 = quantization_keys.reshape(_NLV, Hsz)
    gn = _GN if _GN is not None else _gumbel_noise_graph()

    pad = lambda e: jnp.pad(e, ((0, _NEP - _NE), (0, 0)))
    e0t = pad(embed0)
    e1t = pad(embed1)
    e2t = pad(embed2)

    att3, gidx8, extra = _tc_call(state.reshape(bsz, Hsz), keys3,
                                  in_proj_weight, bq, bk, gn, e0t, e1t, e2t)

    att = att3.reshape(1, bsz, _NLV)
    table = jnp.concatenate(
        [embed0.reshape(-1), embed1.reshape(-1),
         embed2.reshape(-1)]).reshape(-1, 64)
    gidx3 = gidx8[:, :4].reshape(32, 4, 128)
    out_rows = _sc_gather(table, gidx3)
    out = out_rows.reshape(bsz, T, Hsz)
    return out, extra[0, 0], att
